# baseline (speedup 1.0000x reference)
import functools

import jax
import jax.numpy as jnp
from jax import lax
from jax.experimental import pallas as pl
from jax.experimental.pallas import tpu as pltpu
from jax.experimental.pallas import tpu_sc as plsc

_VOCAB = 1000000
_EMB = 64
_PAD = 128
_BATCH = 16384
_HIST = 200
_NW = 32
_LB = 128
_NBC = _BATCH // _LB
_BCW = _NBC // _NW
_NBLK = _BCW * _HIST
_NBUF = 2
_IBUF = 4


def _make_lookup():
    mesh = plsc.VectorSubcoreMesh(core_axis_name="c", subcore_axis_name="s")

    @functools.partial(
        pl.kernel,
        mesh=mesh,
        out_type=jax.ShapeDtypeStruct((_HIST, _EMB, _BATCH), jnp.float32),
        scratch_types=[
            pltpu.VMEM((_LB,), jnp.int32),
            pltpu.VMEM((_LB,), jnp.int32),
            pltpu.VMEM((_LB,), jnp.int32),
            pltpu.VMEM((_LB,), jnp.int32),
            pltpu.VMEM((_LB, _PAD), jnp.float32),
            pltpu.VMEM((_LB, _PAD), jnp.float32),
            pltpu.VMEM((_EMB, _LB), jnp.float32),
            pltpu.VMEM((_EMB, _LB), jnp.float32),
            pltpu.SemaphoreType.DMA,
            pltpu.SemaphoreType.DMA,
            pltpu.SemaphoreType.DMA,
            pltpu.SemaphoreType.DMA,
            pltpu.SemaphoreType.DMA,
            pltpu.SemaphoreType.DMA,
            pltpu.SemaphoreType.DMA,
            pltpu.SemaphoreType.DMA,
        ],
        compiler_params=pltpu.CompilerParams(use_tc_tiling_on_sc=True,
                                             needs_layout_passes=False),
    )
    def lookup(idx_hbm, table_hbm, out_hbm,
               i0, i1, i2, i3, r0, r1, t0, t1,
               si0, si1, si2, si3, sg0, sg1, ss0, ss1):
        idx_v = (i0, i1, i2, i3)
        rows_v = (r0, r1)
        tout_v = (t0, t1)
        sem_i = (si0, si1, si2, si3)
        sem_g = (sg0, sg1)
        sem_s = (ss0, ss1)
        wid = lax.axis_index("s") * 2 + lax.axis_index("c")
        bc0 = wid * _BCW

        bvecs = [jnp.arange(16, dtype=jnp.int32) + 16 * bg for bg in range(8)]

        def idx_src(j):
            h = j % _HIST
            bc = bc0 + j // _HIST
            return idx_hbm.at[h, pl.ds(bc * _LB, _LB)]

        def start_idx(j, s):
            pltpu.async_copy(idx_src(j), idx_v[s], sem_i[s])

        def wait_idx(j, s):
            pltpu.make_async_copy(idx_src(j), idx_v[s], sem_i[s]).wait()

        def out_dst(j):
            h = j % _HIST
            bc = bc0 + j // _HIST
            return out_hbm.at[h, pl.ds(0, _EMB), pl.ds(bc * _LB, _LB)]

        for s in range(_IBUF - 1):
            start_idx(s, s)
        wait_idx(0, 0)
        pltpu.async_copy(table_hbm.at[idx_v[0]], rows_v[0], sem_g[0])

        def body(j0, carry):
            for k in range(_IBUF):
                j = j0 * _IBUF + k
                b = k % _NBUF
                nb = (k + 1) % _NBUF
                pltpu.make_async_copy(table_hbm.at[idx_v[k]],
                                      rows_v[b], sem_g[b]).wait()
                @pl.when(j + 1 < _NBLK)
                def _():
                    wait_idx(j + 1, (k + 1) % _IBUF)
                    @pl.when(j >= 1)
                    def _():
                        pltpu.make_async_copy(tout_v[nb], out_dst(0),
                                              sem_s[nb]).wait()
                    pltpu.async_copy(table_hbm.at[idx_v[(k + 1) % _IBUF]],
                                     rows_v[nb], sem_g[nb])
                @pl.when(j + _IBUF - 1 < _NBLK)
                def _():
                    start_idx(j + _IBUF - 1, (k + _IBUF - 1) % _IBUF)
                for d in range(_EMB):
                    dvec = jnp.full((16,), d, dtype=jnp.int32)
                    for bg in range(8):
                        vals = plsc.load_gather(rows_v[b],
                                                [bvecs[bg], dvec])
                        tout_v[b][d, pl.ds(16 * bg, 16)] = vals
                pltpu.async_copy(tout_v[b], out_dst(j), sem_s[b])
            return carry

        lax.fori_loop(0, _NBLK // _IBUF, body, 0)

        for b in range(_NBUF):
            pltpu.make_async_copy(tout_v[b], out_dst(0), sem_s[b]).wait()

    return lookup


_lookup = _make_lookup()


@jax.jit
def kernel(indices, table):
    table_p = jnp.pad(table, ((0, 0), (0, _PAD - _EMB)))
    out_t = _lookup(indices.T, table_p)
    return jnp.transpose(out_t, (2, 0, 1))


# speedup vs baseline: 1.3941x; 1.3941x over previous
import functools

import jax
import jax.numpy as jnp
from jax import lax
from jax.experimental import pallas as pl
from jax.experimental.pallas import tpu as pltpu
from jax.experimental.pallas import tpu_sc as plsc

_VOCAB = 1000000
_EMB = 64
_PAD = 128
_BATCH = 16384
_HIST = 200
_NW = 32
_LB = 128
_NBC = _BATCH // _LB
_BCW = _NBC // _NW
_NBUF = 2


def _make_lookup():
    mesh = plsc.VectorSubcoreMesh(core_axis_name="c", subcore_axis_name="s")

    @functools.partial(
        pl.kernel,
        mesh=mesh,
        out_type=jax.ShapeDtypeStruct((_HIST, _EMB, _BATCH), jnp.float32),
        scratch_types=[
            pltpu.VMEM((_HIST, _LB), jnp.int32),
            pltpu.VMEM((_LB, _PAD), jnp.float32),
            pltpu.VMEM((_LB, _PAD), jnp.float32),
            pltpu.VMEM((_EMB, _LB), jnp.float32),
            pltpu.VMEM((_EMB, _LB), jnp.float32),
            pltpu.SemaphoreType.DMA,
            pltpu.SemaphoreType.DMA,
            pltpu.SemaphoreType.DMA,
            pltpu.SemaphoreType.DMA,
            pltpu.SemaphoreType.DMA,
        ],
        compiler_params=pltpu.CompilerParams(use_tc_tiling_on_sc=True,
                                             needs_layout_passes=False),
    )
    def lookup(idx_hbm, table_hbm, out_hbm,
               idx_v, r0, r1, t0, t1,
               sem_i, sg0, sg1, ss0, ss1):
        rows_v = (r0, r1)
        tout_v = (t0, t1)
        sem_g = (sg0, sg1)
        sem_s = (ss0, ss1)
        wid = lax.axis_index("s") * 2 + lax.axis_index("c")
        bc0 = wid * _BCW

        dvecs = [jnp.arange(16, dtype=jnp.int32) + 16 * dg for dg in range(4)]

        def transpose_block(b):
            for bl in range(0, _LB, 2):
                pairs = [(bl + i, dg) for i in range(2) for dg in range(4)]
                vals = [rows_v[b][p, pl.ds(16 * dg, 16)] for p, dg in pairs]
                for (p, dg), v in zip(pairs, vals):
                    plsc.store_scatter(
                        tout_v[b],
                        [dvecs[dg], jnp.full((16,), p, dtype=jnp.int32)], v)

        def stripe(s, carry):
            bc = bc0 + s
            pltpu.sync_copy(
                idx_hbm.at[pl.ds(0, _HIST), pl.ds(bc * _LB, _LB)], idx_v)

            def out_dst(h):
                return out_hbm.at[h, pl.ds(0, _EMB), pl.ds(bc * _LB, _LB)]

            pltpu.async_copy(table_hbm.at[idx_v.at[0]], rows_v[0], sem_g[0])

            def body(h0, carry2):
                for k in range(_NBUF):
                    h = h0 * _NBUF + k
                    b = k
                    nb = (k + 1) % _NBUF
                    pltpu.make_async_copy(table_hbm.at[idx_v.at[h]],
                                          rows_v[b], sem_g[b]).wait()
                    @pl.when(h + 1 < _HIST)
                    def _():
                        @pl.when(h >= 1)
                        def _():
                            pltpu.make_async_copy(tout_v[nb], out_dst(0),
                                                  sem_s[nb]).wait()
                        pltpu.async_copy(table_hbm.at[idx_v.at[h + 1]],
                                         rows_v[nb], sem_g[nb])
                    transpose_block(b)
                    pltpu.async_copy(tout_v[b], out_dst(h), sem_s[b])
                return carry2

            lax.fori_loop(0, _HIST // _NBUF, body, 0)

            for b in range(_NBUF):
                pltpu.make_async_copy(tout_v[b], out_dst(0), sem_s[b]).wait()
            return carry

        lax.fori_loop(0, _BCW, stripe, 0)

    return lookup


_lookup = _make_lookup()


@jax.jit
def kernel(indices, table):
    table_p = jnp.pad(table, ((0, 0), (0, _PAD - _EMB)))
    out_t = _lookup(indices.T, table_p)
    return jnp.transpose(out_t, (2, 0, 1))


# speedup vs baseline: 2.2176x; 1.5908x over previous
import functools

import jax
import jax.numpy as jnp
from jax import lax
from jax.experimental import pallas as pl
from jax.experimental.pallas import tpu as pltpu
from jax.experimental.pallas import tpu_sc as plsc

_VOCAB = 1000000
_EMB = 64
_PAD = 128
_BATCH = 16384
_HIST = 200
_NW = 32
_LB = 128
_NBC = _BATCH // _LB
_BCW = _NBC // _NW
_NBUF = 2


def _make_lookup():
    mesh = plsc.VectorSubcoreMesh(core_axis_name="c", subcore_axis_name="s")

    @functools.partial(
        pl.kernel,
        mesh=mesh,
        out_type=jax.ShapeDtypeStruct((_HIST, _EMB, _BATCH), jnp.float32),
        scratch_types=[
            pltpu.VMEM((_HIST, _LB), jnp.int32),
            pltpu.VMEM((_LB, _PAD), jnp.float32),
            pltpu.VMEM((_LB, _PAD), jnp.float32),
            pltpu.VMEM((_EMB, _LB), jnp.float32),
            pltpu.VMEM((_EMB, _LB), jnp.float32),
            pltpu.SemaphoreType.DMA,
            pltpu.SemaphoreType.DMA,
            pltpu.SemaphoreType.DMA,
            pltpu.SemaphoreType.DMA,
            pltpu.SemaphoreType.DMA,
        ],
        compiler_params=pltpu.CompilerParams(use_tc_tiling_on_sc=True,
                                             needs_layout_passes=False),
    )
    def lookup(idx_hbm, table_hbm, out_hbm,
               idx_v, r0, r1, t0, t1,
               sem_i, sg0, sg1, ss0, ss1):
        rows_v = (r0, r1)
        tout_v = (t0, t1)
        sem_g = (sg0, sg1)
        sem_s = (ss0, ss1)
        wid = lax.axis_index("s") * 2 + lax.axis_index("c")
        bc0 = wid * _BCW

        dvecs = [jnp.arange(16, dtype=jnp.int32) + 16 * dg for dg in range(4)]

        lanes = jnp.arange(16, dtype=jnp.int32)

        def transpose_block(b):
            for bl in range(0, _LB, 2):
                pairs = [(bl + i, dg) for i in range(2) for dg in range(4)]
                bvecs = {p: (lanes + p) & (_LB - 1) for p in (bl, bl + 1)}
                vals = [plsc.load_gather(rows_v[b], [bvecs[p], dvecs[dg]])
                        for p, dg in pairs]
                for (p, dg), v in zip(pairs, vals):
                    plsc.store_scatter(tout_v[b], [dvecs[dg], bvecs[p]], v)

        def stripe(s, carry):
            bc = bc0 + s
            pltpu.sync_copy(
                idx_hbm.at[pl.ds(0, _HIST), pl.ds(bc * _LB, _LB)], idx_v)

            def out_dst(h):
                return out_hbm.at[h, pl.ds(0, _EMB), pl.ds(bc * _LB, _LB)]

            pltpu.async_copy(table_hbm.at[idx_v.at[0]], rows_v[0], sem_g[0])

            def body(h0, carry2):
                for k in range(_NBUF):
                    h = h0 * _NBUF + k
                    b = k
                    nb = (k + 1) % _NBUF
                    pltpu.make_async_copy(table_hbm.at[idx_v.at[h]],
                                          rows_v[b], sem_g[b]).wait()
                    @pl.when(h + 1 < _HIST)
                    def _():
                        @pl.when(h >= 1)
                        def _():
                            pltpu.make_async_copy(
                                tout_v[nb], out_dst(0), sem_s[nb]).wait()
                        pltpu.async_copy(table_hbm.at[idx_v.at[h + 1]],
                                         rows_v[nb], sem_g[nb])
                    transpose_block(b)
                    pltpu.async_copy(tout_v[b], out_dst(h), sem_s[b])
                return carry2

            lax.fori_loop(0, _HIST // _NBUF, body, 0)

            for b in range(_NBUF):
                pltpu.make_async_copy(tout_v[b], out_dst(0),
                                      sem_s[b]).wait()
            return carry

        lax.fori_loop(0, _BCW, stripe, 0)

    return lookup


_lookup = _make_lookup()


@jax.jit
def kernel(indices, table):
    table_p = jnp.pad(table, ((0, 0), (0, _PAD - _EMB)))
    out_t = _lookup(indices.T, table_p)
    return jnp.transpose(out_t, (2, 0, 1))
